# baseline (speedup 1.0000x reference)
import functools

import jax
import jax.numpy as jnp
from jax import lax
from jax.experimental import pallas as pl
from jax.experimental.pallas import tpu as pltpu
from jax.experimental.pallas import tpu_sc as plsc

NC = 2
NS = 16
NW = NC * NS
LANES = 16
CH = 128
NBUF = 2

VBLK = 128
PR = VBLK // 2


@functools.partial(jax.jit, static_argnames=("v", "d"))
def _pack(w0t, w1t, v, d):
    n_blocks = (v + VBLK - 1) // VBLK
    rows = n_blocks * PR
    mesh = plsc.VectorSubcoreMesh(
        core_axis_name="c", subcore_axis_name="s",
        num_cores=NC, num_subcores=NS)

    @functools.partial(
        pl.kernel,
        mesh=mesh,
        compiler_params=pltpu.CompilerParams(
            needs_layout_passes=False, disable_bounds_checks=True),
        out_type=jax.ShapeDtypeStruct((rows, 128), jnp.float32),
        scratch_types=[
            pltpu.VMEM((NBUF, d, VBLK), jnp.float32),
            pltpu.VMEM((NBUF, d, VBLK), jnp.float32),
            pltpu.VMEM((NBUF, PR, 128), jnp.float32),
            pltpu.SemaphoreType.DMA((NBUF,)),
            pltpu.SemaphoreType.DMA((NBUF,)),
        ],
    )
    def body(w0_hbm, w1_hbm, wr_hbm, a0, a1, ob, semi, semo):
        wid = lax.axis_index("s") * NC + lax.axis_index("c")
        n_w = 244 + jnp.where(wid < n_blocks - 244 * NW, 1, 0)

        def stage(t, b):
            c = wid + NW * t
            pltpu.async_copy(
                w0_hbm.at[:, pl.ds(c * VBLK, VBLK)], a0.at[b], semi.at[b])
            pltpu.async_copy(
                w1_hbm.at[:, pl.ds(c * VBLK, VBLK)], a1.at[b], semi.at[b])

        stage(0, 0)

        @pl.when(n_w > 1)
        def _():
            stage(1, 1)

        def block_body(t, carry):
            b = t % NBUF
            c = wid + NW * t
            pltpu.make_async_copy(
                w0_hbm.at[:, pl.ds(0, VBLK)], a0.at[b], semi.at[b]).wait()
            pltpu.make_async_copy(
                w0_hbm.at[:, pl.ds(0, VBLK)], a1.at[b], semi.at[b]).wait()

            @pl.when(t >= NBUF)
            def _():
                pltpu.make_async_copy(
                    ob.at[b], wr_hbm.at[pl.ds(0, PR)], semo.at[b]).wait()

            for g in range(VBLK // LANES):
                gi = lax.iota(jnp.int32, LANES) + g * LANES
                rvec = gi >> 1
                cbase = (gi & 1) * 64

                def pack_d(dd, c2):
                    pvt = a0[b, dd, pl.ds(g * LANES, LANES)]
                    plsc.store_scatter(ob.at[b], [rvec, cbase + dd], pvt)
                    qvt = a1[b, dd, pl.ds(g * LANES, LANES)]
                    plsc.store_scatter(
                        ob.at[b], [rvec, cbase + (32 + dd)], qvt)
                    return c2

                lax.fori_loop(0, d, pack_d, 0)

            @pl.when(t + NBUF < n_w)
            def _():
                stage(t + NBUF, b)

            pltpu.async_copy(
                ob.at[b], wr_hbm.at[pl.ds(c * PR, PR)], semo.at[b])
            return carry

        lax.fori_loop(0, n_w, block_body, 0)

        for b in range(NBUF):
            @pl.when(n_w > b)
            def _():
                pltpu.make_async_copy(
                    ob.at[b], wr_hbm.at[pl.ds(0, PR)], semo.at[b]).wait()

    return body(w0t, w1t)


@functools.partial(jax.jit, static_argnames=("f", "bt", "d"))
def _gather(xT, wr, f, bt, d):
    per_w = f * bt // NW
    mesh = plsc.VectorSubcoreMesh(
        core_axis_name="c", subcore_axis_name="s",
        num_cores=NC, num_subcores=NS)

    @functools.partial(
        pl.kernel,
        mesh=mesh,
        compiler_params=pltpu.CompilerParams(needs_layout_passes=False),
        out_type=jax.ShapeDtypeStruct((f, d, bt * CH), jnp.float32),
        scratch_types=[
            pltpu.VMEM((NBUF, CH), jnp.int32),
            pltpu.VMEM((NBUF, CH), jnp.int32),
            pltpu.VMEM((NBUF, CH, 128), jnp.float32),
            pltpu.VMEM((NBUF, d, CH), jnp.float32),
            pltpu.SemaphoreType.DMA((NBUF,)),
            pltpu.SemaphoreType.DMA((NBUF,)),
            pltpu.SemaphoreType.DMA((NBUF,)),
        ],
    )
    def body(x_hbm, wr_hbm, out_hbm, idxq, qoff, r, o, semi, semg, semo):
        wid = lax.axis_index("s") * NC + lax.axis_index("c")
        blk0 = wid * per_w

        def stage_idx(k, b):
            blk = blk0 + k
            pltpu.async_copy(
                x_hbm.at[blk // bt, pl.ds((blk % bt) * CH, CH)],
                idxq.at[b], semi.at[b])

        def fire_gather(b):
            pltpu.make_async_copy(
                x_hbm.at[0, pl.ds(0, CH)], idxq.at[b], semi.at[b]).wait()

            def split(g, carry):
                vv = idxq[b, pl.ds(g * LANES, LANES)]
                qoff[b, pl.ds(g * LANES, LANES)] = (vv & 1) * 64
                idxq[b, pl.ds(g * LANES, LANES)] = vv >> 1
                return carry

            lax.fori_loop(0, CH // LANES, split, 0)
            pltpu.async_copy(wr_hbm.at[idxq.at[b]], r.at[b], semg.at[b])

        for b in range(NBUF):
            stage_idx(b, b)
        fire_gather(0)

        def block_body(k, carry):
            b = k % NBUF
            blk = blk0 + k
            pltpu.make_async_copy(
                wr_hbm.at[idxq.at[b]], r.at[b], semg.at[b]).wait()

            @pl.when(k + 1 < per_w)
            def _():
                fire_gather((k + 1) % NBUF)

            @pl.when(k >= NBUF)
            def _():
                pltpu.make_async_copy(
                    o.at[b], out_hbm.at[0, pl.ds(0, d), pl.ds(0, CH)],
                    semo.at[b]).wait()

            def group_body(g, carry2):
                rws = lax.iota(jnp.int32, LANES) + g * LANES
                cols = qoff[b, pl.ds(g * LANES, LANES)]
                for dd in range(32):
                    v0 = plsc.load_gather(r.at[b], [rws, cols + dd])
                    v1 = plsc.load_gather(r.at[b], [rws, cols + (32 + dd)])
                    o[b, dd, pl.ds(g * LANES, LANES)] = v0 + v1
                return carry2

            lax.fori_loop(0, CH // LANES, group_body, 0)

            pltpu.async_copy(
                o.at[b],
                out_hbm.at[blk // bt, pl.ds(0, d), pl.ds((blk % bt) * CH, CH)],
                semo.at[b])

            @pl.when(k + NBUF < per_w)
            def _():
                stage_idx(k + NBUF, b)
            return carry

        lax.fori_loop(0, per_w, block_body, 0)

        for b in range(NBUF):
            pltpu.make_async_copy(
                o.at[b], out_hbm.at[0, pl.ds(0, d), pl.ds(0, CH)],
                semo.at[b]).wait()

    return body(xT, wr)


def kernel(x, W0, W1):
    bsz, f = x.shape
    v, d = W0.shape
    bt = bsz // CH
    xT = x.T
    w0t = W0.T
    w1t = W1.T
    wr = _pack(w0t, w1t, v, d)
    out_t = _gather(xT, wr, f, bt, d)
    return out_t.transpose(2, 0, 1)


# speedup vs baseline: 1.6854x; 1.6854x over previous
import functools

import jax
import jax.numpy as jnp
from jax import lax
from jax.experimental import pallas as pl
from jax.experimental.pallas import tpu as pltpu
from jax.experimental.pallas import tpu_sc as plsc

NC = 2
NS = 16
NW = NC * NS
LANES = 16
CH = 128

VBLK = 128
PR = VBLK // 2


@functools.partial(jax.jit, static_argnames=("v", "d"))
def _pack(w0t, w1t, v, d):
    n_blocks = (v + VBLK - 1) // VBLK
    rows = n_blocks * PR
    base_w = n_blocks // NW
    t_max = base_w + 1
    mesh = plsc.VectorSubcoreMesh(
        core_axis_name="c", subcore_axis_name="s",
        num_cores=NC, num_subcores=NS)

    @functools.partial(
        pl.kernel,
        mesh=mesh,
        compiler_params=pltpu.CompilerParams(
            needs_layout_passes=False, disable_bounds_checks=True),
        out_type=jax.ShapeDtypeStruct((rows, 128), jnp.float32),
        scratch_types=[
            pltpu.VMEM((2, d, VBLK), jnp.float32),
            pltpu.VMEM((2, d, VBLK), jnp.float32),
            pltpu.VMEM((2, PR, 128), jnp.float32),
            pltpu.SemaphoreType.DMA((2,)),
            pltpu.SemaphoreType.DMA((2,)),
        ],
    )
    def body(w0_hbm, w1_hbm, wr_hbm, a0, a1, ob, semi, semo):
        wid = lax.axis_index("s") * NC + lax.axis_index("c")
        n_w = base_w + jnp.where(wid < n_blocks - base_w * NW, 1, 0)

        def stage(t, b):
            c = wid + NW * t
            pltpu.async_copy(
                w0_hbm.at[:, pl.ds(c * VBLK, VBLK)], a0.at[b], semi.at[b])
            pltpu.async_copy(
                w1_hbm.at[:, pl.ds(c * VBLK, VBLK)], a1.at[b], semi.at[b])

        stage(0, 0)

        def block_body(t, carry):
            b = t % 2
            active = t < n_w

            @pl.when(active)
            def _():
                pltpu.make_async_copy(
                    w0_hbm.at[:, pl.ds(0, VBLK)], a0.at[b],
                    semi.at[b]).wait()
                pltpu.make_async_copy(
                    w0_hbm.at[:, pl.ds(0, VBLK)], a1.at[b],
                    semi.at[b]).wait()

            @pl.when(t + 1 < n_w)
            def _():
                stage(t + 1, (t + 1) % 2)

            @pl.when((t >= 1) & (t - 1 < n_w))
            def _():
                c = wid + NW * (t - 1)
                pltpu.async_copy(
                    ob.at[(t - 1) % 2], wr_hbm.at[pl.ds(c * PR, PR)],
                    semo.at[(t - 1) % 2])

            @pl.when((t >= 2) & active)
            def _():
                pltpu.make_async_copy(
                    ob.at[b], wr_hbm.at[pl.ds(0, PR)], semo.at[b]).wait()

            @pl.when(active)
            def _():
                @functools.partial(plsc.parallel_loop, 0, 256, unroll=4)
                def pack_i(i):
                    g = i >> 5
                    dd = i & 31
                    gi = lax.iota(jnp.int32, LANES) + g * LANES
                    rvec = gi >> 1
                    cb = (gi & 1) * 64
                    pvt = a0[b, dd, pl.ds(g * LANES, LANES)]
                    plsc.store_scatter(ob.at[b], [rvec, cb + dd], pvt)
                    qvt = a1[b, dd, pl.ds(g * LANES, LANES)]
                    plsc.store_scatter(ob.at[b], [rvec, cb + 32 + dd], qvt)

            return carry

        lax.fori_loop(0, t_max, block_body, 0)

        @pl.when(n_w == t_max)
        def _():
            c = wid + NW * (t_max - 1)
            pltpu.async_copy(
                ob.at[(t_max - 1) % 2], wr_hbm.at[pl.ds(c * PR, PR)],
                semo.at[(t_max - 1) % 2])

        for s in range(2):
            pltpu.make_async_copy(
                ob.at[s], wr_hbm.at[pl.ds(0, PR)], semo.at[s]).wait()

    return body(w0t, w1t)


NB = 3


@functools.partial(jax.jit, static_argnames=("f", "bt", "d"))
def _gather(xT, wr, f, bt, d):
    per_w = f * bt // NW
    mesh = plsc.VectorSubcoreMesh(
        core_axis_name="c", subcore_axis_name="s",
        num_cores=NC, num_subcores=NS)

    @functools.partial(
        pl.kernel,
        mesh=mesh,
        compiler_params=pltpu.CompilerParams(needs_layout_passes=False),
        out_type=jax.ShapeDtypeStruct((f, d, bt * CH), jnp.float32),
        scratch_types=[
            pltpu.VMEM((NB, CH), jnp.int32),
            pltpu.VMEM((NB, CH), jnp.int32),
            pltpu.VMEM((NB, CH, 128), jnp.float32),
            pltpu.VMEM((NB, d, CH), jnp.float32),
            pltpu.SemaphoreType.DMA((NB,)),
            pltpu.SemaphoreType.DMA((NB,)),
            pltpu.SemaphoreType.DMA((NB,)),
        ],
    )
    def body(x_hbm, wr_hbm, out_hbm, idxq, qoff, r, o, semi, semg, semo):
        wid = lax.axis_index("s") * NC + lax.axis_index("c")
        blk0 = wid * per_w

        def stage_idx(k, b):
            blk = blk0 + k
            pltpu.async_copy(
                x_hbm.at[blk // bt, pl.ds((blk % bt) * CH, CH)],
                idxq.at[b], semi.at[b])

        def fire_gather(b):
            pltpu.make_async_copy(
                x_hbm.at[0, pl.ds(0, CH)], idxq.at[b], semi.at[b]).wait()

            def split(g, carry):
                vv = idxq[b, pl.ds(g * LANES, LANES)]
                qoff[b, pl.ds(g * LANES, LANES)] = (vv & 1) * 64
                idxq[b, pl.ds(g * LANES, LANES)] = vv >> 1
                return carry

            lax.fori_loop(0, CH // LANES, split, 0)
            pltpu.async_copy(wr_hbm.at[idxq.at[b]], r.at[b], semg.at[b])

        for b in range(NB):
            stage_idx(b, b)
        fire_gather(0)

        def block_body(k, carry):
            b = k % NB
            pltpu.make_async_copy(
                wr_hbm.at[idxq.at[b]], r.at[b], semg.at[b]).wait()

            @pl.when(k + 1 < per_w)
            def _():
                fire_gather((k + 1) % NB)

            @pl.when(k >= 1)
            def _():
                blkp = blk0 + k - 1
                pltpu.async_copy(
                    o.at[(k - 1) % NB],
                    out_hbm.at[blkp // bt, pl.ds(0, d),
                               pl.ds((blkp % bt) * CH, CH)],
                    semo.at[(k - 1) % NB])

            @pl.when(k >= NB)
            def _():
                pltpu.make_async_copy(
                    o.at[b], out_hbm.at[0, pl.ds(0, d), pl.ds(0, CH)],
                    semo.at[b]).wait()

            def extract_i(i, c2):
                g = i >> 5
                dd = i & 31
                rws = lax.iota(jnp.int32, LANES) + g * LANES
                cols = qoff[b, pl.ds(g * LANES, LANES)]
                v0 = plsc.load_gather(r.at[b], [rws, cols + dd])
                v1 = plsc.load_gather(r.at[b], [rws, cols + 32 + dd])
                o[b, dd, pl.ds(g * LANES, LANES)] = v0 + v1
                return c2

            lax.fori_loop(0, 256, extract_i, 0)

            @pl.when(k + NB < per_w)
            def _():
                stage_idx(k + NB, b)
            return carry

        lax.fori_loop(0, per_w, block_body, 0)

        blkl = blk0 + per_w - 1
        pltpu.async_copy(
            o.at[(per_w - 1) % NB],
            out_hbm.at[blkl // bt, pl.ds(0, d),
                       pl.ds((blkl % bt) * CH, CH)],
            semo.at[(per_w - 1) % NB])
        for b in range(NB):
            pltpu.make_async_copy(
                o.at[b], out_hbm.at[0, pl.ds(0, d), pl.ds(0, CH)],
                semo.at[b]).wait()

    return body(xT, wr)


def kernel(x, W0, W1):
    bsz, f = x.shape
    v, d = W0.shape
    bt = bsz // CH
    xT = x.T
    w0t = W0.T
    w1t = W1.T
    wr = _pack(w0t, w1t, v, d)
    out_t = _gather(xT, wr, f, bt, d)
    return out_t.transpose(2, 0, 1)
